# Initial kernel scaffold; baseline (speedup 1.0000x reference)
import jax
import jax.numpy as jnp
from jax.experimental import pallas as pl


def kernel(input_ids, table, Wproj):
    raise NotImplementedError("write your pallas kernel here")


# speedup vs baseline: 1.5298x; 1.5298x over previous
import functools

import jax
import jax.numpy as jnp
from jax import lax
from jax.experimental import pallas as pl
from jax.experimental.pallas import tpu as pltpu
from jax.experimental.pallas import tpu_sc as plsc

BIGRAM_VOCAB = 100000
HID = 128
MODEL_DIM = 1024
BATCH = 4
SEQLEN = 4096
TOK = BATCH * SEQLEN

NC, NS = 2, 16
NW = NC * NS
CHUNK = TOK // NW
GSTREAM = 128
NG = CHUNK // GSTREAM
VECS = CHUNK // 16


@functools.partial(
    pl.kernel,
    mesh=plsc.VectorSubcoreMesh(core_axis_name="c", subcore_axis_name="s"),
    out_type=jax.ShapeDtypeStruct((TOK, HID), jnp.float32),
    scratch_types=[
        pltpu.VMEM((CHUNK,), jnp.int32),
        pltpu.VMEM((CHUNK,), jnp.int32),
        pltpu.VMEM((NG, GSTREAM), jnp.int32),
        pltpu.VMEM((CHUNK, HID), jnp.float32),
        pltpu.SemaphoreType.DMA,
    ],
)
def _sc_gather(cur_hbm, prev_hbm, table_hbm, h_hbm, cur_v, prev_v, idx_v,
               rows_v, sem):
    wid = lax.axis_index("s") * NC + lax.axis_index("c")
    base = wid * CHUNK
    pltpu.sync_copy(cur_hbm.at[pl.ds(base, CHUNK)], cur_v)
    pltpu.sync_copy(prev_hbm.at[pl.ds(base, CHUNK)], prev_v)
    for i in range(VECS):
        cur = cur_v[pl.ds(i * 16, 16)]
        prev = prev_v[pl.ds(i * 16, 16)]
        h = lax.rem(prev * 3 + cur, jnp.int32(100000))
        idx_v[i * 16 // GSTREAM, pl.ds((i * 16) % GSTREAM, 16)] = h
    copies = [
        pltpu.async_copy(table_hbm.at[idx_v.at[jnp.int32(j)]],
                         rows_v.at[pl.ds(j * GSTREAM, GSTREAM)], sem)
        for j in range(NG)
    ]
    for cp in copies:
        cp.wait()
    pltpu.sync_copy(rows_v, h_hbm.at[pl.ds(base, CHUNK)])


def _proj_body(h_ref, w_ref, o_ref):
    o_ref[...] = lax.dot_general(
        h_ref[...], w_ref[...], (((1,), (1,)), ((), ())),
        preferred_element_type=jnp.float32)


_ROWS_BLK = 1024


def _tc_project(h, Wproj):
    return pl.pallas_call(
        _proj_body,
        grid=(TOK // _ROWS_BLK,),
        in_specs=[
            pl.BlockSpec((_ROWS_BLK, HID), lambda i: (i, jnp.int32(0))),
            pl.BlockSpec((MODEL_DIM, HID),
                         lambda i: (jnp.int32(0), jnp.int32(0))),
        ],
        out_specs=pl.BlockSpec((_ROWS_BLK, MODEL_DIM),
                               lambda i: (i, jnp.int32(0))),
        out_shape=jax.ShapeDtypeStruct((TOK, MODEL_DIM), jnp.float32),
    )(h, Wproj)


def kernel(input_ids, table, Wproj):
    ids32 = input_ids.astype(jnp.int32)
    prev32 = jnp.concatenate(
        [jnp.zeros((BATCH, 1), jnp.int32), ids32[:, :-1]], axis=1)
    h = _sc_gather(ids32.reshape(TOK), prev32.reshape(TOK), table)
    out = _tc_project(h, Wproj)
    return out.reshape(BATCH, SEQLEN, MODEL_DIM)
